# Initial kernel scaffold; baseline (speedup 1.0000x reference)
import jax
import jax.numpy as jnp
from jax.experimental import pallas as pl


def kernel(xyz, sparse_xyz, sparse_frame):
    raise NotImplementedError("write your pallas kernel here")


# speedup vs baseline: 33.8308x; 33.8308x over previous
import jax
import jax.numpy as jnp
from jax import lax
from jax.experimental import pallas as pl

_BN = 256


def _fused_body(xq_ref, sx_ref, ft_ref, out_ref):
    bn = xq_ref.shape[0]
    s = sx_ref.shape[1]

    xq = xq_ref[...]
    sx = sx_ref[...]
    xs = lax.dot_general(
        xq.astype(jnp.bfloat16), sx.astype(jnp.bfloat16),
        (((1,), (0,)), ((), ())),
        preferred_element_type=jnp.float32)
    xq2 = xq[:, 0:1] ** 2 + xq[:, 1:2] ** 2 + xq[:, 2:3] ** 2
    sx2 = sx[0:1, :] ** 2 + sx[1:2, :] ** 2 + sx[2:3, :] ** 2
    dn = (-2.0 * xs + xq2) + sx2

    dd = (xq[:, 0:1] - sx[0:1, :]) ** 2
    dd = dd + (xq[:, 1:2] - sx[1:2, :]) ** 2
    dd = dd + (xq[:, 2:3] - sx[2:3, :]) ** 2

    iota = lax.broadcasted_iota(jnp.int32, (bn, s), 1)
    big = jnp.float32(jnp.inf)

    def extract_min(dcur):
        m = jnp.min(dcur, axis=1, keepdims=True)
        idx = jnp.min(jnp.where(dcur == m, iota, s), axis=1,
                      keepdims=True)
        sel = jnp.where(iota == idx, dd, big)
        val = jnp.min(sel, axis=1, keepdims=True)
        dnext = jnp.where(iota == idx, big, dcur)
        return val, dnext

    m1, dn = extract_min(dn)
    m2, dn = extract_min(dn)
    m3, _ = extract_min(dn)

    inv1 = 1.0 / jnp.maximum(jnp.sqrt(m1), 1e-10)
    inv2 = 1.0 / jnp.maximum(jnp.sqrt(m2), 1e-10)
    inv3 = 1.0 / jnp.maximum(jnp.sqrt(m3), 1e-10)
    norm = inv1 + inv2 + inv3
    w = jnp.concatenate([inv1, inv2, inv3], axis=1) / norm

    out_ref[...] = lax.dot_general(
        ft_ref[...], w, (((1,), (1,)), ((), ())),
        preferred_element_type=jnp.float32)


def kernel(xyz, sparse_xyz, sparse_frame):
    b, c, n = xyz.shape
    s = sparse_xyz.shape[2]
    xq = jnp.transpose(xyz[0])
    sx = sparse_xyz[0]
    ft = jnp.transpose(sparse_frame[0])

    out = pl.pallas_call(
        _fused_body,
        grid=(n // _BN,),
        in_specs=[
            pl.BlockSpec((_BN, c), lambda i: (i, 0)),
            pl.BlockSpec((c, s), lambda i: (0, 0)),
            pl.BlockSpec((s, c), lambda i: (0, 0)),
        ],
        out_specs=pl.BlockSpec((s, _BN), lambda i: (0, i)),
        out_shape=jax.ShapeDtypeStruct((s, n), jnp.float32),
    )(xq, sx, ft)
    return out[None]
